# baseline (speedup 1.0000x reference)
import jax
import jax.numpy as jnp
from jax import lax
from jax.experimental import pallas as pl
from jax.experimental.pallas import tpu as pltpu
from jax.experimental.pallas import tpu_sc as plsc

SF = 2
B, C, T = 64, 256, 4096
RI = 8
RO = 4
ROT = RO * T
NBLK = T // 16
NC, NS = 2, 16
SAMPLES_PER_W = B // (NC * NS)
ICS = C // RI
N_IC = SAMPLES_PER_W * ICS
LOG_ICS = 5


def _compute_half(g, half, b0, in_v, out_v, k_v, iota):
    bi = b0 + lax.shift_right_logical(g, LOG_ICS)
    k_vec = plsc.load_gather(k_v, [jnp.full((16,), bi, jnp.int32)]) - SF
    base = iota - k_vec
    for r in range(RO):
        rsplat = jnp.full((16,), RO * half + r, jnp.int32)
        row0 = r * T
        colL = jnp.where(base < 0, -base, base)
        out_v[pl.ds(row0, 16)] = plsc.load_gather(in_v, [rsplat, colL])

        @plsc.parallel_loop(1, NBLK - 1, unroll=8, carry=base + 16)
        def blk(j, col, row0=row0, rsplat=rsplat):
            out_v[pl.ds(row0 + j * 16, 16)] = plsc.load_gather(
                in_v, [rsplat, col])
            return col + 16

        colR = base + (NBLK - 1) * 16
        colR = jnp.where(colR > T - 1, 2 * (T - 1) - colR, colR)
        out_v[pl.ds(row0 + (NBLK - 1) * 16, 16)] = plsc.load_gather(
            in_v, [rsplat, colR])


def _body(x_hbm, k_hbm, out_hbm, k_v, in0, in1, outb0, outb1,
          sin0, sin1, sout0, sout1):
    wid = lax.axis_index("s") * NC + lax.axis_index("c")
    b0 = wid * SAMPLES_PER_W
    pltpu.sync_copy(k_hbm, k_v)
    iota = lax.iota(jnp.int32, 16)

    def start_in(g, buf, sem):
        b = b0 + lax.shift_right_logical(g, LOG_ICS)
        c0 = pl.multiple_of(lax.shift_left(g & (ICS - 1), 3), 8)
        pltpu.make_async_copy(x_hbm.at[b, pl.ds(c0, RI), :], buf, sem).start()

    def wait_in(buf, sem):
        pltpu.make_async_copy(x_hbm.at[0, pl.ds(0, RI), :], buf, sem).wait()

    def start_out(g, half, buf, sem):
        b = b0 + lax.shift_right_logical(g, LOG_ICS)
        c0 = lax.shift_left(g & (ICS - 1), 3) + RO * half
        for r in range(RO):
            pltpu.make_async_copy(
                buf.at[pl.ds(r * T, T)], out_hbm.at[b, c0 + r, :], sem).start()

    def wait_out(buf, sem):
        for r in range(RO):
            pltpu.make_async_copy(
                buf.at[pl.ds(r * T, T)], out_hbm.at[0, 0, :], sem).wait()

    ins = ((in0, sin0), (in1, sin1))
    outs = ((outb0, sout0), (outb1, sout1))

    def do_chunk(g, q, first):
        iv, isem = ins[q]
        wait_in(iv, isem)
        for half in range(2):
            ov, osem = outs[half]
            if not first:
                wait_out(ov, osem)
            _compute_half(g, half, b0, iv, ov, k_v, iota)
            start_out(g, half, ov, osem)

    start_in(jnp.int32(0), in0, sin0)
    start_in(jnp.int32(1), in1, sin1)
    do_chunk(jnp.int32(0), 0, True)
    start_in(jnp.int32(2), in0, sin0)

    def steady(gg, carry):
        for q, dg in ((1, 1), (0, 2)):
            g = 2 * gg + dg
            do_chunk(g, q, False)
            start_in(g + 2, ins[q][0], ins[q][1])
        return carry

    lax.fori_loop(0, 30, steady, 0)

    do_chunk(jnp.int32(61), 1, False)
    start_in(jnp.int32(63), in1, sin1)
    do_chunk(jnp.int32(62), 0, False)
    do_chunk(jnp.int32(63), 1, False)
    for ov, osem in outs:
        wait_out(ov, osem)


@jax.jit
def kernel(x, k_list):
    mesh = plsc.VectorSubcoreMesh(core_axis_name="c", subcore_axis_name="s")
    run = pl.kernel(
        _body,
        out_type=jax.ShapeDtypeStruct((B, C, T), jnp.float32),
        mesh=mesh,
        scratch_types=[
            pltpu.VMEM((B,), jnp.int32),
            pltpu.VMEM((RI, T), jnp.float32),
            pltpu.VMEM((RI, T), jnp.float32),
            pltpu.VMEM((ROT,), jnp.float32),
            pltpu.VMEM((ROT,), jnp.float32),
            pltpu.SemaphoreType.DMA,
            pltpu.SemaphoreType.DMA,
            pltpu.SemaphoreType.DMA,
            pltpu.SemaphoreType.DMA,
        ],
        compiler_params=pltpu.CompilerParams(needs_layout_passes=False),
    )
    return run(x, k_list.astype(jnp.int32))


# speedup vs baseline: 1.0151x; 1.0151x over previous
import jax
import jax.numpy as jnp
from jax import lax
from jax.experimental import pallas as pl
from jax.experimental.pallas import tpu as pltpu
from jax.experimental.pallas import tpu_sc as plsc

SF = 2
B, C, T = 64, 256, 4096
R = 4
RT = R * T
NBLK = T // 16
NC, NS = 2, 16
SAMPLES_PER_W = B // (NC * NS)
CPS = C // R
LOG_CPS = 6
N_CHUNK = SAMPLES_PER_W * CPS


def _compute_chunk(ci, b0, in_v, out_v, k_v, iota):
    bi = b0 + lax.shift_right_logical(ci, LOG_CPS)
    k_vec = plsc.load_gather(k_v, [jnp.full((16,), bi, jnp.int32)]) - SF
    base = iota - k_vec
    for r in range(R):
        row0 = r * T
        colL = jnp.where(base < 0, -base, base)
        out_v[pl.ds(row0, 16)] = plsc.load_gather(in_v, [colL + row0])

        @plsc.parallel_loop(1, NBLK - 1, unroll=16, carry=base + row0 + 16)
        def blk(j, idx, row0=row0):
            out_v[pl.ds(row0 + j * 16, 16)] = plsc.load_gather(in_v, [idx])
            return idx + 16

        colR = base + (NBLK - 1) * 16
        colR = jnp.where(colR > T - 1, 2 * (T - 1) - colR, colR)
        out_v[pl.ds(row0 + (NBLK - 1) * 16, 16)] = plsc.load_gather(
            in_v, [colR + row0])


def _body(x_hbm, k_hbm, out_hbm, k_v, in0, in1, in2, out0, out1, out2,
          sin0, sin1, sin2, sout0, sout1, sout2):
    wid = lax.axis_index("s") * NC + lax.axis_index("c")
    b0 = wid * SAMPLES_PER_W
    pltpu.sync_copy(k_hbm, k_v)
    iota = lax.iota(jnp.int32, 16)

    def src_at(ci):
        b = b0 + lax.shift_right_logical(ci, LOG_CPS)
        c0 = lax.shift_left(ci & (CPS - 1), 2)
        return b, c0

    def start_in(ci, buf, sem):
        b, c0 = src_at(ci)
        for r in range(R):
            pltpu.make_async_copy(
                x_hbm.at[b, c0 + r, :], buf.at[pl.ds(r * T, T)], sem).start()

    def wait_in(buf, sem):
        for r in range(R):
            pltpu.make_async_copy(
                x_hbm.at[0, 0, :], buf.at[pl.ds(r * T, T)], sem).wait()

    def start_out(ci, buf, sem):
        b, c0 = src_at(ci)
        for r in range(R):
            pltpu.make_async_copy(
                buf.at[pl.ds(r * T, T)], out_hbm.at[b, c0 + r, :], sem).start()

    def wait_out(buf, sem):
        for r in range(R):
            pltpu.make_async_copy(
                buf.at[pl.ds(r * T, T)], out_hbm.at[0, 0, :], sem).wait()

    def compute(ci, in_v, out_v):
        _compute_chunk(ci, b0, in_v, out_v, k_v, iota)

    bufs = ((in0, sin0, out0, sout0), (in1, sin1, out1, sout1),
            (in2, sin2, out2, sout2))

    for p in range(3):
        start_in(jnp.int32(p), bufs[p][0], bufs[p][1])
    for p in range(3):
        iv, isem, ov, osem = bufs[p]
        wait_in(iv, isem)
        compute(jnp.int32(p), iv, ov)
        start_out(jnp.int32(p), ov, osem)
        start_in(jnp.int32(p + 3), iv, isem)

    def steady(g, carry):
        for p in range(3):
            iv, isem, ov, osem = bufs[p]
            ci = 3 * g + p
            wait_in(iv, isem)
            wait_out(ov, osem)
            compute(ci, iv, ov)
            start_out(ci, ov, osem)
            start_in(ci + 3, iv, isem)
        return carry

    lax.fori_loop(1, 41, steady, 0)

    for i, p in enumerate((0, 1, 2, 0, 1)):
        iv, isem, ov, osem = bufs[p]
        ci = jnp.int32(123 + i)
        wait_in(iv, isem)
        wait_out(ov, osem)
        compute(ci, iv, ov)
        start_out(ci, ov, osem)
        if i < 2:
            start_in(ci + 3, iv, isem)
    for p in (2, 0, 1):
        _, _, ov, osem = bufs[p]
        wait_out(ov, osem)


@jax.jit
def kernel(x, k_list):
    mesh = plsc.VectorSubcoreMesh(core_axis_name="c", subcore_axis_name="s")
    run = pl.kernel(
        _body,
        out_type=jax.ShapeDtypeStruct((B, C, T), jnp.float32),
        mesh=mesh,
        scratch_types=[
            pltpu.VMEM((B,), jnp.int32),
            pltpu.VMEM((RT,), jnp.float32),
            pltpu.VMEM((RT,), jnp.float32),
            pltpu.VMEM((RT,), jnp.float32),
            pltpu.VMEM((RT,), jnp.float32),
            pltpu.VMEM((RT,), jnp.float32),
            pltpu.VMEM((RT,), jnp.float32),
            pltpu.SemaphoreType.DMA,
            pltpu.SemaphoreType.DMA,
            pltpu.SemaphoreType.DMA,
            pltpu.SemaphoreType.DMA,
            pltpu.SemaphoreType.DMA,
            pltpu.SemaphoreType.DMA,
        ],
        compiler_params=pltpu.CompilerParams(needs_layout_passes=False),
    )
    return run(x, k_list.astype(jnp.int32))
